# baseline (speedup 1.0000x reference)
import functools

import jax
import jax.numpy as jnp
from jax.experimental import pallas as pl
from jax.experimental.pallas import tpu as pltpu


def _dma_body(B, S, t_ref, x_ref, p_ref, o_ref, xsem, psem):
    copies = []
    for b in range(B):
        cx = pltpu.make_async_copy(
            x_ref.at[b], o_ref.at[b, pl.ds(0, S), :], xsem.at[b])
        cp = pltpu.make_async_copy(
            p_ref.at[pl.ds(t_ref[b], 1), :], o_ref.at[b, pl.ds(S, 1), :],
            psem.at[b])
        cx.start()
        cp.start()
        copies.append((cx, cp))
    for cx, cp in copies:
        cx.wait()
        cp.wait()


def kernel(x, task_id, prompt):
    B, S, D = x.shape

    grid_spec = pltpu.PrefetchScalarGridSpec(
        num_scalar_prefetch=1,
        grid=(1,),
        in_specs=[
            pl.BlockSpec(memory_space=pl.ANY),
            pl.BlockSpec(memory_space=pl.ANY),
        ],
        out_specs=pl.BlockSpec(memory_space=pl.ANY),
        scratch_shapes=[
            pltpu.SemaphoreType.DMA((B,)),
            pltpu.SemaphoreType.DMA((B,)),
        ],
    )
    out = pl.pallas_call(
        functools.partial(_dma_body, B, S),
        grid_spec=grid_spec,
        out_shape=jax.ShapeDtypeStruct((B, S + 1, D), x.dtype),
    )(task_id, x, prompt)
    return (out, task_id)


# speedup vs baseline: 13.1874x; 13.1874x over previous
import jax
import jax.numpy as jnp
from jax.experimental import pallas as pl
from jax.experimental.pallas import tpu as pltpu

SEQ_BLOCK = 1024


def _body(t_ref, x_ref, p_ref, o_ref):
    s = pl.program_id(1)
    ns = pl.num_programs(1)

    @pl.when(s < ns - 1)
    def _copy():
        o_ref[...] = x_ref[...]

    @pl.when(s == ns - 1)
    def _prompt_row():
        o_ref[0, 0, :] = p_ref[0, 0, :]


def kernel(x, task_id, prompt):
    B, S, D = x.shape
    n_sb = S // SEQ_BLOCK
    prompt3 = prompt.reshape(prompt.shape[0], 1, D)

    grid_spec = pltpu.PrefetchScalarGridSpec(
        num_scalar_prefetch=1,
        grid=(B, n_sb + 1),
        in_specs=[
            pl.BlockSpec((1, SEQ_BLOCK, D),
                         lambda b, s, t: (b, jnp.minimum(s, n_sb - 1), 0)),
            pl.BlockSpec((1, 1, D), lambda b, s, t: (t[b], 0, 0)),
        ],
        out_specs=pl.BlockSpec((1, SEQ_BLOCK, D), lambda b, s, t: (b, s, 0)),
    )
    out = pl.pallas_call(
        _body,
        grid_spec=grid_spec,
        out_shape=jax.ShapeDtypeStruct((B, S + 1, D), x.dtype),
        compiler_params=pltpu.CompilerParams(
            dimension_semantics=("parallel", "arbitrary"),
        ),
    )(task_id, x, prompt3)
    return (out, task_id)
